# baseline (speedup 1.0000x reference)
import functools

import jax
import jax.numpy as jnp
from jax import lax
from jax.experimental import pallas as pl
from jax.experimental.pallas import tpu as pltpu
from jax.experimental.pallas import tpu_sc as plsc

_CUTOFF = 5.0
_NC = 2
_NS = 16
_NW = _NC * _NS
_L = 16

_R_INV = 0
_R_CELL = 9
_R_NRM = 18
_R_CUT = 27
_R_HMC = 28
_R_BAD = 31
_NCONST = 32


def _round_bf16(x):
    u = lax.bitcast_convert_type(x, jnp.uint32)
    lsb = lax.shift_right_logical(u, jnp.uint32(16)) & jnp.uint32(1)
    r = (u + jnp.uint32(0x7FFF) + lsb) & jnp.uint32(0xFFFF0000)
    return lax.bitcast_convert_type(r, jnp.float32)


def _group_body(g, abase, kstride, inp_v, const_v, buf):
    f32 = jnp.float32

    def crow(i):
        return const_v[i, :]

    a0 = abase + g * _L
    col = (g // 8) * 1024 + (g % 8) * _L
    px = inp_v[0, pl.ds(a0, _L)]
    py = inp_v[1, pl.ds(a0, _L)]
    pz = inp_v[2, pl.ds(a0, _L)]
    bad = crow(_R_BAD)

    frac = []
    for d in range(3):
        fr = (px * crow(_R_INV + d) + py * crow(_R_INV + 3 + d)
              + pz * crow(_R_INV + 6 + d))
        t = fr.astype(jnp.int32).astype(f32)
        fl = jnp.where(t > fr, t - 1.0, t)
        frac.append(_round_bf16(fr - fl))
    A = [[crow(_R_CELL + 3 * r + d) for d in range(3)] for r in range(3)]
    w = [frac[0] * A[0][d] + frac[1] * A[1][d] + frac[2] * A[2][d]
         for d in range(3)]
    wb = [_round_bf16(w[d]) for d in range(3)]
    lo, hi = [], []
    cut = crow(_R_CUT)
    for d in range(3):
        nc = (wb[0] * crow(_R_NRM + 3 * d) + wb[1] * crow(_R_NRM + 3 * d + 1)
              + wb[2] * crow(_R_NRM + 3 * d + 2))
        lo.append(nc <= cut)
        hi.append(nc >= crow(_R_HMC + d))
    w = [w[d] + bad for d in range(3)]

    vel = [inp_v[3, pl.ds(a0, _L)] + bad,
           inp_v[4, pl.ds(a0, _L)] + bad,
           inp_v[5, pl.ds(a0, _L)] + bad]
    ms = inp_v[6, pl.ds(a0, _L)] + bad

    axv = [None, lo[0], hi[0]]
    ayv = [None, lo[1], hi[1]]
    azv = [None, lo[2], hi[2]]

    k = 0
    for i in range(3):
        if i == 0:
            xi = w
        elif i == 1:
            xi = [w[c] + A[0][c] for c in range(3)]
        else:
            xi = [w[c] - A[0][c] for c in range(3)]
        for j in range(3):
            if j == 0:
                xij = xi
            elif j == 1:
                xij = [xi[c] + A[1][c] for c in range(3)]
            else:
                xij = [xi[c] - A[1][c] for c in range(3)]
            mij = None
            for b in (axv[i], ayv[j]):
                if b is not None:
                    mij = b if mij is None else (mij & b)
            for l in range(3):
                if l == 0:
                    p3 = xij
                elif l == 1:
                    p3 = [xij[c] + A[2][c] for c in range(3)]
                else:
                    p3 = [xij[c] - A[2][c] for c in range(3)]
                m = mij
                if azv[l] is not None:
                    m = azv[l] if m is None else (m & azv[l])
                base = k * kstride + col
                if m is None:
                    for c in range(3):
                        buf[pl.ds(base + c * 128, _L)] = p3[c]
                        buf[pl.ds(base + (3 + c) * 128, _L)] = vel[c]
                    buf[pl.ds(base + 6 * 128, _L)] = ms
                else:
                    mf = m.astype(f32)
                    for c in range(3):
                        buf[pl.ds(base + c * 128, _L)] = p3[c] * mf
                        buf[pl.ds(base + (3 + c) * 128, _L)] = vel[c] * mf
                    buf[pl.ds(base + 6 * 128, _L)] = ms * mf
                k += 1


def _sc_body(apt, inp_hbm, const_hbm, out_hbm, inp_v, const_v, buf0, buf1,
             sem0, sem1):
    wid = lax.axis_index("s") * _NC + lax.axis_index("c")
    base = wid * apt
    pltpu.sync_copy(inp_hbm.at[:, pl.ds(base, apt)], inp_v)
    pltpu.sync_copy(const_hbm, const_v)

    nblk = apt // 128
    j0 = wid * nblk

    @plsc.parallel_loop(0, 24, 1, unroll=4)
    def _r1(g):
        _group_body(g, 0, 3072, inp_v, const_v, buf0)

    handles = []
    for k in range(27):
        handles.append(pltpu.async_copy(
            buf0.at[pl.ds(k * 3072, 3072)],
            out_hbm.at[pl.ds((k * 128 + j0) * 1024, 3072)], sem0))

    @plsc.parallel_loop(0, 8, 1, unroll=4)
    def _r2(g):
        _group_body(g, 384, 1024, inp_v, const_v, buf1)

    for k in range(27):
        handles.append(pltpu.async_copy(
            buf1.at[pl.ds(k * 1024, 1024)],
            out_hbm.at[pl.ds((k * 128 + j0 + 3) * 1024, 1024)], sem1))
    for h in handles:
        h.wait()


def kernel(positions, cell, types, masses, velocities):
    del types
    f32 = jnp.float32
    n = positions.shape[0]
    apt = n // _NW

    a, b, c = cell[0, 0], cell[0, 1], cell[0, 2]
    d, e, f_ = cell[1, 0], cell[1, 1], cell[1, 2]
    g, h, i = cell[2, 0], cell[2, 1], cell[2, 2]
    ca, cb, cc = e * i - f_ * h, f_ * g - d * i, d * h - e * g
    det = a * ca + b * cb + c * cc
    adj = jnp.stack([
        jnp.stack([ca, c * h - b * i, b * f_ - c * e]),
        jnp.stack([cb, a * i - c * g, c * d - a * f_]),
        jnp.stack([cc, b * g - a * h, a * e - b * d]),
    ])
    inv_cell = adj / det
    recip = inv_cell.T
    norms = jnp.linalg.norm(recip, axis=1)
    heights = 1.0 / norms
    normals = recip / norms[:, None]
    bad = jnp.where(heights.min() >= _CUTOFF, 0.0, jnp.nan).astype(f32)

    def bf(x):
        u = lax.bitcast_convert_type(x.astype(f32), jnp.uint32)
        lsb = lax.shift_right_logical(u, jnp.uint32(16)) & jnp.uint32(1)
        r = (u + jnp.uint32(0x7FFF) + lsb) & jnp.uint32(0xFFFF0000)
        return lax.bitcast_convert_type(r, f32)

    inp = jnp.concatenate(
        [bf(positions.T), velocities.T, masses[None, :]], axis=0)
    cvals = jnp.concatenate([
        bf(inv_cell).reshape(-1), bf(cell).reshape(-1),
        bf(normals).reshape(-1),
        jnp.array([_CUTOFF], dtype=f32), heights - _CUTOFF,
        bad[None],
    ]).astype(f32)
    const = jnp.tile(cvals[:, None], (1, _L))

    ntiles = n // 128
    mesh = plsc.VectorSubcoreMesh(core_axis_name="c", subcore_axis_name="s",
                                  num_cores=_NC, num_subcores=_NS)
    sc_call = pl.kernel(
        functools.partial(_sc_body, apt),
        out_type=jax.ShapeDtypeStruct((27 * ntiles * 1024,), f32),
        mesh=mesh,
        compiler_params=pltpu.CompilerParams(needs_layout_passes=False),
        scratch_types=[
            pltpu.VMEM((7, apt), f32),
            pltpu.VMEM((_NCONST, _L), f32),
            pltpu.VMEM((27 * 3072,), f32),
            pltpu.VMEM((27 * 1024,), f32),
            pltpu.SemaphoreType.DMA,
            pltpu.SemaphoreType.DMA,
        ],
    )
    out = sc_call(inp, const)
    return (out.reshape(27, ntiles, 8, 128)
            .transpose(1, 3, 0, 2)
            .reshape(n, 27, 8)[:, :, :7])


# speedup vs baseline: 1.0096x; 1.0096x over previous
import functools

import jax
import jax.numpy as jnp
from jax import lax
from jax.experimental import pallas as pl
from jax.experimental.pallas import tpu as pltpu
from jax.experimental.pallas import tpu_sc as plsc

_CUTOFF = 5.0
_NC = 2
_NS = 16
_NW = _NC * _NS
_L = 16

_R_INV = 0
_R_CELL = 9
_R_NRM = 18
_R_CUT = 27
_R_HMC = 28
_R_BAD = 31
_NCONST = 32


def _round_bf16(x):
    u = lax.bitcast_convert_type(x, jnp.uint32)
    lsb = lax.shift_right_logical(u, jnp.uint32(16)) & jnp.uint32(1)
    r = (u + jnp.uint32(0x7FFF) + lsb) & jnp.uint32(0xFFFF0000)
    return lax.bitcast_convert_type(r, jnp.float32)


def _group_body(g, abase, kstride, inp_v, const_v, buf):
    f32 = jnp.float32

    def crow(i):
        return const_v[i, :]

    a0 = abase + g * _L
    col = (g // 8) * 1024 + (g % 8) * _L
    px = inp_v[0, pl.ds(a0, _L)]
    py = inp_v[1, pl.ds(a0, _L)]
    pz = inp_v[2, pl.ds(a0, _L)]
    bad = crow(_R_BAD)

    frac = []
    for d in range(3):
        fr = (px * crow(_R_INV + d) + py * crow(_R_INV + 3 + d)
              + pz * crow(_R_INV + 6 + d))
        t = fr.astype(jnp.int32).astype(f32)
        fl = jnp.where(t > fr, t - 1.0, t)
        frac.append(_round_bf16(fr - fl))
    A = [[crow(_R_CELL + 3 * r + d) for d in range(3)] for r in range(3)]
    w = [frac[0] * A[0][d] + frac[1] * A[1][d] + frac[2] * A[2][d]
         for d in range(3)]
    wb = [_round_bf16(w[d]) for d in range(3)]
    lo, hi = [], []
    cut = crow(_R_CUT)
    for d in range(3):
        nc = (wb[0] * crow(_R_NRM + 3 * d) + wb[1] * crow(_R_NRM + 3 * d + 1)
              + wb[2] * crow(_R_NRM + 3 * d + 2))
        lo.append(nc <= cut)
        hi.append(nc >= crow(_R_HMC + d))
    w = [w[d] + bad for d in range(3)]

    vel = [inp_v[3, pl.ds(a0, _L)] + bad,
           inp_v[4, pl.ds(a0, _L)] + bad,
           inp_v[5, pl.ds(a0, _L)] + bad]
    ms = inp_v[6, pl.ds(a0, _L)] + bad

    axv = [None, lo[0], hi[0]]
    ayv = [None, lo[1], hi[1]]
    azv = [None, lo[2], hi[2]]

    k = 0
    for i in range(3):
        if i == 0:
            xi = w
        elif i == 1:
            xi = [w[c] + A[0][c] for c in range(3)]
        else:
            xi = [w[c] - A[0][c] for c in range(3)]
        for j in range(3):
            if j == 0:
                xij = xi
            elif j == 1:
                xij = [xi[c] + A[1][c] for c in range(3)]
            else:
                xij = [xi[c] - A[1][c] for c in range(3)]
            mij = None
            for b in (axv[i], ayv[j]):
                if b is not None:
                    mij = b if mij is None else (mij & b)
            for l in range(3):
                if l == 0:
                    p3 = xij
                elif l == 1:
                    p3 = [xij[c] + A[2][c] for c in range(3)]
                else:
                    p3 = [xij[c] - A[2][c] for c in range(3)]
                m = mij
                if azv[l] is not None:
                    m = azv[l] if m is None else (m & azv[l])
                base = k * kstride + col
                if m is None:
                    for c in range(3):
                        buf[pl.ds(base + c * 128, _L)] = p3[c]
                        buf[pl.ds(base + (3 + c) * 128, _L)] = vel[c]
                    buf[pl.ds(base + 6 * 128, _L)] = ms
                else:
                    mf = m.astype(f32)
                    for c in range(3):
                        buf[pl.ds(base + c * 128, _L)] = p3[c] * mf
                        buf[pl.ds(base + (3 + c) * 128, _L)] = vel[c] * mf
                    buf[pl.ds(base + 6 * 128, _L)] = ms * mf
                k += 1


def _sc_body(apt, inp_hbm, const_hbm, out_hbm, inp_v, const_v, buf0, buf1,
             sem0, sem1):
    wid = lax.axis_index("s") * _NC + lax.axis_index("c")
    base = wid * apt
    pltpu.sync_copy(inp_hbm.at[:, pl.ds(base, apt)], inp_v)
    pltpu.sync_copy(const_hbm, const_v)

    nblk = apt // 128
    j0 = wid * nblk

    @plsc.parallel_loop(0, 24, 1, unroll=2)
    def _r1(g):
        _group_body(g, 0, 3072, inp_v, const_v, buf0)

    handles = []
    for k in range(27):
        handles.append(pltpu.async_copy(
            buf0.at[pl.ds(k * 3072, 3072)],
            out_hbm.at[pl.ds((k * 128 + j0) * 1024, 3072)], sem0))

    @plsc.parallel_loop(0, 8, 1, unroll=2)
    def _r2(g):
        _group_body(g, 384, 1024, inp_v, const_v, buf1)

    for k in range(27):
        handles.append(pltpu.async_copy(
            buf1.at[pl.ds(k * 1024, 1024)],
            out_hbm.at[pl.ds((k * 128 + j0 + 3) * 1024, 1024)], sem1))
    for h in handles:
        h.wait()


def kernel(positions, cell, types, masses, velocities):
    del types
    f32 = jnp.float32
    n = positions.shape[0]
    apt = n // _NW

    a, b, c = cell[0, 0], cell[0, 1], cell[0, 2]
    d, e, f_ = cell[1, 0], cell[1, 1], cell[1, 2]
    g, h, i = cell[2, 0], cell[2, 1], cell[2, 2]
    ca, cb, cc = e * i - f_ * h, f_ * g - d * i, d * h - e * g
    det = a * ca + b * cb + c * cc
    adj = jnp.stack([
        jnp.stack([ca, c * h - b * i, b * f_ - c * e]),
        jnp.stack([cb, a * i - c * g, c * d - a * f_]),
        jnp.stack([cc, b * g - a * h, a * e - b * d]),
    ])
    inv_cell = adj / det
    recip = inv_cell.T
    norms = jnp.linalg.norm(recip, axis=1)
    heights = 1.0 / norms
    normals = recip / norms[:, None]
    bad = jnp.where(heights.min() >= _CUTOFF, 0.0, jnp.nan).astype(f32)

    def bf(x):
        u = lax.bitcast_convert_type(x.astype(f32), jnp.uint32)
        lsb = lax.shift_right_logical(u, jnp.uint32(16)) & jnp.uint32(1)
        r = (u + jnp.uint32(0x7FFF) + lsb) & jnp.uint32(0xFFFF0000)
        return lax.bitcast_convert_type(r, f32)

    inp = jnp.concatenate(
        [bf(positions.T), velocities.T, masses[None, :]], axis=0)
    cvals = jnp.concatenate([
        bf(inv_cell).reshape(-1), bf(cell).reshape(-1),
        bf(normals).reshape(-1),
        jnp.array([_CUTOFF], dtype=f32), heights - _CUTOFF,
        bad[None],
    ]).astype(f32)
    const = jnp.tile(cvals[:, None], (1, _L))

    ntiles = n // 128
    mesh = plsc.VectorSubcoreMesh(core_axis_name="c", subcore_axis_name="s",
                                  num_cores=_NC, num_subcores=_NS)
    sc_call = pl.kernel(
        functools.partial(_sc_body, apt),
        out_type=jax.ShapeDtypeStruct((27 * ntiles * 1024,), f32),
        mesh=mesh,
        compiler_params=pltpu.CompilerParams(needs_layout_passes=False),
        scratch_types=[
            pltpu.VMEM((7, apt), f32),
            pltpu.VMEM((_NCONST, _L), f32),
            pltpu.VMEM((27 * 3072,), f32),
            pltpu.VMEM((27 * 1024,), f32),
            pltpu.SemaphoreType.DMA,
            pltpu.SemaphoreType.DMA,
        ],
    )
    out = sc_call(inp, const)
    return (out.reshape(27, ntiles, 8, 128)
            .transpose(1, 3, 0, 2)
            .reshape(n, 27, 8)[:, :, :7])


# speedup vs baseline: 1.5060x; 1.4916x over previous
import functools

import jax
import jax.numpy as jnp
from jax import lax
from jax.experimental import pallas as pl
from jax.experimental.pallas import tpu as pltpu
from jax.experimental.pallas import tpu_sc as plsc

_CUTOFF = 5.0
_NC = 2
_NS = 16
_NW = _NC * _NS
_L = 16

_R_INV = 0
_R_CELL = 9
_R_NRM = 18
_R_CUT = 27
_R_HMC = 28
_R_BAD = 31
_NCONST = 32


def _round_bf16(x):
    u = lax.bitcast_convert_type(x, jnp.uint32)
    lsb = lax.shift_right_logical(u, jnp.uint32(16)) & jnp.uint32(1)
    r = (u + jnp.uint32(0x7FFF) + lsb) & jnp.uint32(0xFFFF0000)
    return lax.bitcast_convert_type(r, jnp.float32)


def _group_body(g, abase, kstride, inp_v, const_v, buf):
    f32 = jnp.float32

    def crow(i):
        return const_v[i, :]

    a0 = abase + g * _L
    col = (g // 8) * 1024 + (g % 8) * _L
    px = inp_v[0, pl.ds(a0, _L)]
    py = inp_v[1, pl.ds(a0, _L)]
    pz = inp_v[2, pl.ds(a0, _L)]
    bad = crow(_R_BAD)

    frac = []
    for d in range(3):
        fr = (px * crow(_R_INV + d) + py * crow(_R_INV + 3 + d)
              + pz * crow(_R_INV + 6 + d))
        t = fr.astype(jnp.int32).astype(f32)
        fl = jnp.where(t > fr, t - 1.0, t)
        frac.append(_round_bf16(fr - fl))
    A = [[crow(_R_CELL + 3 * r + d) for d in range(3)] for r in range(3)]
    w = [frac[0] * A[0][d] + frac[1] * A[1][d] + frac[2] * A[2][d]
         for d in range(3)]
    wb = [_round_bf16(w[d]) for d in range(3)]
    lo, hi = [], []
    cut = crow(_R_CUT)
    for d in range(3):
        nc = (wb[0] * crow(_R_NRM + 3 * d) + wb[1] * crow(_R_NRM + 3 * d + 1)
              + wb[2] * crow(_R_NRM + 3 * d + 2))
        lo.append(nc <= cut)
        hi.append(nc >= crow(_R_HMC + d))
    w = [w[d] + bad for d in range(3)]

    vel = [inp_v[3, pl.ds(a0, _L)] + bad,
           inp_v[4, pl.ds(a0, _L)] + bad,
           inp_v[5, pl.ds(a0, _L)] + bad]
    ms = inp_v[6, pl.ds(a0, _L)] + bad

    axv = [None, lo[0], hi[0]]
    ayv = [None, lo[1], hi[1]]
    azv = [None, lo[2], hi[2]]

    k = 0
    for i in range(3):
        if i == 0:
            xi = w
        elif i == 1:
            xi = [w[c] + A[0][c] for c in range(3)]
        else:
            xi = [w[c] - A[0][c] for c in range(3)]
        for j in range(3):
            if j == 0:
                xij = xi
            elif j == 1:
                xij = [xi[c] + A[1][c] for c in range(3)]
            else:
                xij = [xi[c] - A[1][c] for c in range(3)]
            mij = None
            for b in (axv[i], ayv[j]):
                if b is not None:
                    mij = b if mij is None else (mij & b)
            for l in range(3):
                if l == 0:
                    p3 = xij
                elif l == 1:
                    p3 = [xij[c] + A[2][c] for c in range(3)]
                else:
                    p3 = [xij[c] - A[2][c] for c in range(3)]
                m = mij
                if azv[l] is not None:
                    m = azv[l] if m is None else (m & azv[l])
                base = k * kstride + col
                if m is None:
                    for c in range(3):
                        buf[pl.ds(base + c * 128, _L)] = p3[c]
                        buf[pl.ds(base + (3 + c) * 128, _L)] = vel[c]
                    buf[pl.ds(base + 6 * 128, _L)] = ms
                else:
                    mf = m.astype(f32)
                    for c in range(3):
                        buf[pl.ds(base + c * 128, _L)] = p3[c] * mf
                        buf[pl.ds(base + (3 + c) * 128, _L)] = vel[c] * mf
                    buf[pl.ds(base + 6 * 128, _L)] = ms * mf
                k += 1


def _sc_body(apt, inp_hbm, const_hbm, out_hbm, inp_v, const_v, buf0, buf1,
             sem0, sem1):
    wid = lax.axis_index("s") * _NC + lax.axis_index("c")
    base = wid * apt
    pltpu.sync_copy(inp_hbm.at[:, pl.ds(base, apt)], inp_v)
    pltpu.sync_copy(const_hbm, const_v)

    nblk = apt // 128
    j0 = wid * nblk

    @plsc.parallel_loop(0, 24, 1, unroll=2)
    def _r1(g):
        _group_body(g, 0, 3072, inp_v, const_v, buf0)

    handles = []
    for k in range(27):
        handles.append(pltpu.async_copy(
            buf0.at[pl.ds(k * 3072, 3072)],
            out_hbm.at[pl.ds((k * 128 + j0) * 1024, 3072)], sem0))

    @plsc.parallel_loop(0, 8, 1, unroll=2)
    def _r2(g):
        _group_body(g, 384, 1024, inp_v, const_v, buf1)

    for k in range(27):
        handles.append(pltpu.async_copy(
            buf1.at[pl.ds(k * 1024, 1024)],
            out_hbm.at[pl.ds((k * 128 + j0 + 3) * 1024, 1024)], sem1))
    for h in handles:
        h.wait()


def kernel(positions, cell, types, masses, velocities):
    del types
    f32 = jnp.float32
    n = positions.shape[0]
    apt = n // _NW

    r1 = jnp.roll(cell, -1, axis=0)
    r2 = jnp.roll(cell, -2, axis=0)
    cof = (jnp.roll(r1, -1, axis=1) * jnp.roll(r2, -2, axis=1)
           - jnp.roll(r1, -2, axis=1) * jnp.roll(r2, -1, axis=1))
    det = jnp.sum(cell[0, :] * cof[0, :])
    inv_cell = cof.T / det
    recip = inv_cell.T
    norms = jnp.linalg.norm(recip, axis=1)
    heights = 1.0 / norms
    normals = recip / norms[:, None]
    bad = jnp.where(heights.min() >= _CUTOFF, 0.0, jnp.nan).astype(f32)

    def bf(x):
        u = lax.bitcast_convert_type(x.astype(f32), jnp.uint32)
        lsb = lax.shift_right_logical(u, jnp.uint32(16)) & jnp.uint32(1)
        r = (u + jnp.uint32(0x7FFF) + lsb) & jnp.uint32(0xFFFF0000)
        return lax.bitcast_convert_type(r, f32)

    inp = jnp.concatenate(
        [bf(positions.T), velocities.T, masses[None, :]], axis=0)
    cvals = jnp.concatenate([
        bf(inv_cell).reshape(-1), bf(cell).reshape(-1),
        bf(normals).reshape(-1),
        jnp.array([_CUTOFF], dtype=f32), heights - _CUTOFF,
        bad[None],
    ]).astype(f32)
    const = jnp.tile(cvals[:, None], (1, _L))

    ntiles = n // 128
    mesh = plsc.VectorSubcoreMesh(core_axis_name="c", subcore_axis_name="s",
                                  num_cores=_NC, num_subcores=_NS)
    sc_call = pl.kernel(
        functools.partial(_sc_body, apt),
        out_type=jax.ShapeDtypeStruct((27 * ntiles * 1024,), f32),
        mesh=mesh,
        compiler_params=pltpu.CompilerParams(needs_layout_passes=False),
        scratch_types=[
            pltpu.VMEM((7, apt), f32),
            pltpu.VMEM((_NCONST, _L), f32),
            pltpu.VMEM((27 * 3072,), f32),
            pltpu.VMEM((27 * 1024,), f32),
            pltpu.SemaphoreType.DMA,
            pltpu.SemaphoreType.DMA,
        ],
    )
    out = sc_call(inp, const)
    return (out.reshape(27, ntiles, 8, 128)
            .transpose(1, 3, 0, 2)
            .reshape(n, 27, 8)[:, :, :7])


# speedup vs baseline: 1.5826x; 1.0509x over previous
import functools

import jax
import jax.numpy as jnp
from jax import lax
from jax.experimental import pallas as pl
from jax.experimental.pallas import tpu as pltpu
from jax.experimental.pallas import tpu_sc as plsc

_CUTOFF = 5.0
_NC = 2
_NS = 16
_NW = _NC * _NS
_L = 16

_R_INV = 0
_R_CELL = 9
_R_NRM = 18
_R_CUT = 27
_R_HMC = 28
_R_BAD = 31
_NCONST = 32


def _round_bf16(x):
    u = lax.bitcast_convert_type(x, jnp.uint32)
    lsb = lax.shift_right_logical(u, jnp.uint32(16)) & jnp.uint32(1)
    r = (u + jnp.uint32(0x7FFF) + lsb) & jnp.uint32(0xFFFF0000)
    return lax.bitcast_convert_type(r, jnp.float32)


def _group_body(g, abase, kstride, inp_v, const_v, buf):
    f32 = jnp.float32

    def crow(i):
        return const_v[i, :]

    a0 = abase + g * _L
    col = (g // 8) * 1024 + (g % 8) * _L
    px = inp_v[0, pl.ds(a0, _L)]
    py = inp_v[1, pl.ds(a0, _L)]
    pz = inp_v[2, pl.ds(a0, _L)]
    bad = crow(_R_BAD)

    frac = []
    for d in range(3):
        fr = (px * crow(_R_INV + d) + py * crow(_R_INV + 3 + d)
              + pz * crow(_R_INV + 6 + d))
        t = fr.astype(jnp.int32).astype(f32)
        fl = jnp.where(t > fr, t - 1.0, t)
        frac.append(_round_bf16(fr - fl))
    A = [[crow(_R_CELL + 3 * r + d) for d in range(3)] for r in range(3)]
    w = [frac[0] * A[0][d] + frac[1] * A[1][d] + frac[2] * A[2][d]
         for d in range(3)]
    wb = [_round_bf16(w[d]) for d in range(3)]
    lo, hi = [], []
    cut = crow(_R_CUT)
    for d in range(3):
        nc = (wb[0] * crow(_R_NRM + 3 * d) + wb[1] * crow(_R_NRM + 3 * d + 1)
              + wb[2] * crow(_R_NRM + 3 * d + 2))
        lo.append(nc <= cut)
        hi.append(nc >= crow(_R_HMC + d))
    w = [w[d] + bad for d in range(3)]

    vel = [inp_v[3, pl.ds(a0, _L)] + bad,
           inp_v[4, pl.ds(a0, _L)] + bad,
           inp_v[5, pl.ds(a0, _L)] + bad]
    ms = inp_v[6, pl.ds(a0, _L)] + bad

    axv = [None, lo[0], hi[0]]
    ayv = [None, lo[1], hi[1]]
    azv = [None, lo[2], hi[2]]

    k = 0
    for i in range(3):
        if i == 0:
            xi = w
        elif i == 1:
            xi = [w[c] + A[0][c] for c in range(3)]
        else:
            xi = [w[c] - A[0][c] for c in range(3)]
        for j in range(3):
            if j == 0:
                xij = xi
            elif j == 1:
                xij = [xi[c] + A[1][c] for c in range(3)]
            else:
                xij = [xi[c] - A[1][c] for c in range(3)]
            mij = None
            for b in (axv[i], ayv[j]):
                if b is not None:
                    mij = b if mij is None else (mij & b)
            for l in range(3):
                if l == 0:
                    p3 = xij
                elif l == 1:
                    p3 = [xij[c] + A[2][c] for c in range(3)]
                else:
                    p3 = [xij[c] - A[2][c] for c in range(3)]
                m = mij
                if azv[l] is not None:
                    m = azv[l] if m is None else (m & azv[l])
                base = k * kstride + col
                if m is None:
                    for c in range(3):
                        buf[pl.ds(base + c * 128, _L)] = p3[c]
                        buf[pl.ds(base + (3 + c) * 128, _L)] = vel[c]
                    buf[pl.ds(base + 6 * 128, _L)] = ms
                else:
                    mf = m.astype(f32)
                    for c in range(3):
                        buf[pl.ds(base + c * 128, _L)] = p3[c] * mf
                        buf[pl.ds(base + (3 + c) * 128, _L)] = vel[c] * mf
                    buf[pl.ds(base + 6 * 128, _L)] = ms * mf
                k += 1


def _sc_body(apt, inp_hbm, const_hbm, out_hbm, inp_v, const_v, buf0, buf1,
             sem0, sem1):
    wid = lax.axis_index("s") * _NC + lax.axis_index("c")
    base = wid * apt
    pltpu.sync_copy(inp_hbm.at[:, pl.ds(base, apt)], inp_v)
    pltpu.sync_copy(const_hbm, const_v)

    nblk = apt // 128
    j0 = wid * nblk

    @plsc.parallel_loop(0, 24, 1, unroll=2)
    def _r1(g):
        _group_body(g, 0, 3072, inp_v, const_v, buf0)

    handles = []
    for k in range(27):
        handles.append(pltpu.async_copy(
            buf0.at[pl.ds(k * 3072, 3072)],
            out_hbm.at[pl.ds((k * 128 + j0) * 1024, 3072)], sem0))

    @plsc.parallel_loop(0, 8, 1, unroll=2)
    def _r2(g):
        _group_body(g, 384, 1024, inp_v, const_v, buf1)

    for k in range(27):
        handles.append(pltpu.async_copy(
            buf1.at[pl.ds(k * 1024, 1024)],
            out_hbm.at[pl.ds((k * 128 + j0 + 3) * 1024, 1024)], sem1))
    for h in handles:
        h.wait()


def kernel(positions, cell, types, masses, velocities):
    del types
    f32 = jnp.float32
    n = positions.shape[0]
    apt = n // _NW

    inv_cell = jnp.linalg.inv(cell)
    recip = inv_cell.T
    norms = jnp.linalg.norm(recip, axis=1)
    heights = 1.0 / norms
    normals = recip / norms[:, None]
    bad = jnp.where(heights.min() >= _CUTOFF, 0.0, jnp.nan).astype(f32)

    def bf(x):
        u = lax.bitcast_convert_type(x.astype(f32), jnp.uint32)
        lsb = lax.shift_right_logical(u, jnp.uint32(16)) & jnp.uint32(1)
        r = (u + jnp.uint32(0x7FFF) + lsb) & jnp.uint32(0xFFFF0000)
        return lax.bitcast_convert_type(r, f32)

    inp = jnp.concatenate(
        [bf(positions.T), velocities.T, masses[None, :]], axis=0)
    cvals = jnp.concatenate([
        bf(inv_cell).reshape(-1), bf(cell).reshape(-1),
        bf(normals).reshape(-1),
        jnp.array([_CUTOFF], dtype=f32), heights - _CUTOFF,
        bad[None],
    ]).astype(f32)
    const = jnp.tile(cvals[:, None], (1, _L))

    ntiles = n // 128
    mesh = plsc.VectorSubcoreMesh(core_axis_name="c", subcore_axis_name="s",
                                  num_cores=_NC, num_subcores=_NS)
    sc_call = pl.kernel(
        functools.partial(_sc_body, apt),
        out_type=jax.ShapeDtypeStruct((27 * ntiles * 1024,), f32),
        mesh=mesh,
        compiler_params=pltpu.CompilerParams(needs_layout_passes=False),
        scratch_types=[
            pltpu.VMEM((7, apt), f32),
            pltpu.VMEM((_NCONST, _L), f32),
            pltpu.VMEM((27 * 3072,), f32),
            pltpu.VMEM((27 * 1024,), f32),
            pltpu.SemaphoreType.DMA,
            pltpu.SemaphoreType.DMA,
        ],
    )
    out = sc_call(inp, const)
    return (out.reshape(27, ntiles, 8, 128)
            .transpose(1, 3, 0, 2)
            .reshape(n, 27, 8)[:, :, :7])


# speedup vs baseline: 1.6835x; 1.0638x over previous
import functools

import jax
import jax.numpy as jnp
from jax import lax
from jax.experimental import pallas as pl
from jax.experimental.pallas import tpu as pltpu
from jax.experimental.pallas import tpu_sc as plsc

_CUTOFF = 5.0
_NC = 2
_NS = 16
_NW = _NC * _NS
_L = 16

_R_INV = 0
_R_CELL = 9
_R_NRM = 18
_R_CUT = 27
_R_HMC = 28
_R_BAD = 31
_NCONST = 32


def _round_bf16(x):
    u = lax.bitcast_convert_type(x, jnp.uint32)
    lsb = lax.shift_right_logical(u, jnp.uint32(16)) & jnp.uint32(1)
    r = (u + jnp.uint32(0x7FFF) + lsb) & jnp.uint32(0xFFFF0000)
    return lax.bitcast_convert_type(r, jnp.float32)


def _group_body(g, abase, kstride, inp_v, const_v, buf):
    f32 = jnp.float32

    def crow(i):
        return const_v[i, :]

    a0 = abase + g * _L
    col = (g // 8) * 1024 + (g % 8) * _L
    px = inp_v[0, pl.ds(a0, _L)]
    py = inp_v[1, pl.ds(a0, _L)]
    pz = inp_v[2, pl.ds(a0, _L)]
    bad = crow(_R_BAD)

    frac = []
    for d in range(3):
        fr = (px * crow(_R_INV + d) + py * crow(_R_INV + 3 + d)
              + pz * crow(_R_INV + 6 + d))
        t = fr.astype(jnp.int32).astype(f32)
        fl = jnp.where(t > fr, t - 1.0, t)
        frac.append(_round_bf16(fr - fl))
    A = [[crow(_R_CELL + 3 * r + d) for d in range(3)] for r in range(3)]
    w = [frac[0] * A[0][d] + frac[1] * A[1][d] + frac[2] * A[2][d]
         for d in range(3)]
    wb = [_round_bf16(w[d]) for d in range(3)]
    lo, hi = [], []
    cut = crow(_R_CUT)
    for d in range(3):
        nc = (wb[0] * crow(_R_NRM + 3 * d) + wb[1] * crow(_R_NRM + 3 * d + 1)
              + wb[2] * crow(_R_NRM + 3 * d + 2))
        lo.append(nc <= cut)
        hi.append(nc >= crow(_R_HMC + d))
    w = [w[d] + bad for d in range(3)]

    vel = [inp_v[3, pl.ds(a0, _L)] + bad,
           inp_v[4, pl.ds(a0, _L)] + bad,
           inp_v[5, pl.ds(a0, _L)] + bad]
    ms = inp_v[6, pl.ds(a0, _L)] + bad

    axv = [None, lo[0], hi[0]]
    ayv = [None, lo[1], hi[1]]
    azv = [None, lo[2], hi[2]]

    k = 0
    for i in range(3):
        if i == 0:
            xi = w
        elif i == 1:
            xi = [w[c] + A[0][c] for c in range(3)]
        else:
            xi = [w[c] - A[0][c] for c in range(3)]
        for j in range(3):
            if j == 0:
                xij = xi
            elif j == 1:
                xij = [xi[c] + A[1][c] for c in range(3)]
            else:
                xij = [xi[c] - A[1][c] for c in range(3)]
            mij = None
            for b in (axv[i], ayv[j]):
                if b is not None:
                    mij = b if mij is None else (mij & b)
            for l in range(3):
                if l == 0:
                    p3 = xij
                elif l == 1:
                    p3 = [xij[c] + A[2][c] for c in range(3)]
                else:
                    p3 = [xij[c] - A[2][c] for c in range(3)]
                m = mij
                if azv[l] is not None:
                    m = azv[l] if m is None else (m & azv[l])
                base = k * kstride + col
                if m is None:
                    for c in range(3):
                        buf[pl.ds(base + c * 128, _L)] = p3[c]
                        buf[pl.ds(base + (3 + c) * 128, _L)] = vel[c]
                    buf[pl.ds(base + 6 * 128, _L)] = ms
                else:
                    mf = m.astype(f32)
                    for c in range(3):
                        buf[pl.ds(base + c * 128, _L)] = p3[c] * mf
                        buf[pl.ds(base + (3 + c) * 128, _L)] = vel[c] * mf
                    buf[pl.ds(base + 6 * 128, _L)] = ms * mf
                k += 1


def _sc_body(apt, inp_hbm, const_hbm, out_hbm, inp_v, const_v, buf0, buf1,
             sem0, sem1):
    wid = lax.axis_index("s") * _NC + lax.axis_index("c")
    base = wid * apt
    pltpu.sync_copy(inp_hbm.at[:, pl.ds(base, apt)], inp_v)
    pltpu.sync_copy(const_hbm, const_v)

    nblk = apt // 128
    j0 = wid * nblk

    @plsc.parallel_loop(0, 24, 1, unroll=3)
    def _r1(g):
        _group_body(g, 0, 3072, inp_v, const_v, buf0)

    handles = []
    for k in range(27):
        handles.append(pltpu.async_copy(
            buf0.at[pl.ds(k * 3072, 3072)],
            out_hbm.at[pl.ds((k * 128 + j0) * 1024, 3072)], sem0))

    @plsc.parallel_loop(0, 8, 1, unroll=2)
    def _r2(g):
        _group_body(g, 384, 1024, inp_v, const_v, buf1)

    for k in range(27):
        handles.append(pltpu.async_copy(
            buf1.at[pl.ds(k * 1024, 1024)],
            out_hbm.at[pl.ds((k * 128 + j0 + 3) * 1024, 1024)], sem1))
    for h in handles:
        h.wait()


def kernel(positions, cell, types, masses, velocities):
    del types
    f32 = jnp.float32
    n = positions.shape[0]
    apt = n // _NW

    inv_cell = jnp.linalg.inv(cell)
    recip = inv_cell.T
    norms = jnp.linalg.norm(recip, axis=1)
    heights = 1.0 / norms
    normals = recip / norms[:, None]
    bad = jnp.where(heights.min() >= _CUTOFF, 0.0, jnp.nan).astype(f32)

    def bf(x):
        u = lax.bitcast_convert_type(x.astype(f32), jnp.uint32)
        lsb = lax.shift_right_logical(u, jnp.uint32(16)) & jnp.uint32(1)
        r = (u + jnp.uint32(0x7FFF) + lsb) & jnp.uint32(0xFFFF0000)
        return lax.bitcast_convert_type(r, f32)

    inp = jnp.concatenate(
        [bf(positions.T), velocities.T, masses[None, :]], axis=0)
    cvals = jnp.concatenate([
        bf(inv_cell).reshape(-1), bf(cell).reshape(-1),
        bf(normals).reshape(-1),
        jnp.array([_CUTOFF], dtype=f32), heights - _CUTOFF,
        bad[None],
    ]).astype(f32)
    const = jnp.tile(cvals[:, None], (1, _L))

    ntiles = n // 128
    mesh = plsc.VectorSubcoreMesh(core_axis_name="c", subcore_axis_name="s",
                                  num_cores=_NC, num_subcores=_NS)
    sc_call = pl.kernel(
        functools.partial(_sc_body, apt),
        out_type=jax.ShapeDtypeStruct((27 * ntiles * 1024,), f32),
        mesh=mesh,
        compiler_params=pltpu.CompilerParams(needs_layout_passes=False),
        scratch_types=[
            pltpu.VMEM((7, apt), f32),
            pltpu.VMEM((_NCONST, _L), f32),
            pltpu.VMEM((27 * 3072,), f32),
            pltpu.VMEM((27 * 1024,), f32),
            pltpu.SemaphoreType.DMA,
            pltpu.SemaphoreType.DMA,
        ],
    )
    out = sc_call(inp, const)
    return (out.reshape(27, ntiles, 8, 128)
            .transpose(1, 3, 0, 2)
            .reshape(n, 27, 8)[:, :, :7])
